# Initial kernel scaffold; baseline (speedup 1.0000x reference)
import jax
import jax.numpy as jnp
from jax.experimental import pallas as pl


def kernel(x, timeline_mask, prototype_vectors, patch_select):
    raise NotImplementedError("write your pallas kernel here")


# speedup vs baseline: 1.0663x; 1.0663x over previous
import functools

import jax
import jax.numpy as jnp
from jax.experimental import pallas as pl
from jax.experimental.pallas import tpu as pltpu

TEMP = 2.0
RADIUS = 16
NEG = -100000.0


def _matcher_kernel(xt_ref, tm_ref, q_ref, ps_ref, act_out, mind_out, idx_out,
                    emb_scr, *, pblk, n_p, t, radius):
    j = pl.program_id(1)

    @pl.when(j == 0)
    def _():
        xv = xt_ref[...]
        nrm = jnp.sqrt(jnp.sum(xv * xv, axis=0, keepdims=True))
        emb_scr[...] = xv / jnp.maximum(nrm, 1e-12)

    q = q_ref[...]
    qn = jnp.sqrt(jnp.sum(q * q, axis=1, keepdims=True))
    qv = q / jnp.maximum(qn, 1e-12)
    dist2 = jnp.dot(qv, emb_scr[...], preferred_element_type=jnp.float32)
    tm = tm_ref[...]
    dist2 = dist2 * tm + (1.0 - tm) * NEG
    dist = dist2.reshape(pblk, n_p, t)

    iota_t3 = jax.lax.broadcasted_iota(jnp.int32, (pblk, n_p, t), 2)
    iota_n2 = jax.lax.broadcasted_iota(jnp.int32, (pblk, n_p), 1)
    tio = jax.lax.broadcasted_iota(jnp.int32, (pblk, 1, t), 2)
    nio3 = jax.lax.broadcasted_iota(jnp.int32, (pblk, n_p, 1), 1)

    act = jnp.ones((pblk, 1, t), jnp.float32)
    sub = jnp.ones((pblk, n_p, 1), jnp.float32)
    adj = jnp.ones((pblk, 1, t), jnp.float32)
    cs, ss, vs = [], [], []
    for it in range(n_p):
        m = act * sub * adj
        dm = dist + (1.0 - m) * NEG
        m1 = jnp.max(dm, axis=2)
        a1 = jnp.min(jnp.where(dm == m1[:, :, None], iota_t3, t), axis=2)
        m2 = jnp.max(m1, axis=1, keepdims=True)
        s = jnp.min(jnp.where(m1 == m2, iota_n2, n_p), axis=1, keepdims=True)
        c = jnp.min(jnp.where(iota_n2 == s, a1, t), axis=1, keepdims=True)
        cb = c[:, :, None]
        act = act * (1.0 - (tio == cb).astype(jnp.float32))
        sub = sub * (1.0 - (nio3 == s[:, :, None]).astype(jnp.float32))
        neigh = jnp.abs(tio - cb) <= radius
        if it > 0:
            adj = jnp.logical_and(neigh, tio > cb).astype(jnp.float32)
        else:
            adj = neigh.astype(jnp.float32)
        cs.append(c)
        ss.append(s)
        vs.append(m2)

    k8 = jax.lax.broadcasted_iota(jnp.int32, (pblk, n_p), 1)
    vals = jnp.zeros((pblk, n_p), jnp.float32)
    idxs = jnp.zeros((pblk, n_p), jnp.float32)
    for i in range(n_p):
        rank = jnp.zeros((pblk, 1), jnp.int32)
        for jj in range(n_p):
            if jj == i:
                continue
            lt = (ss[jj] < ss[i]) | ((ss[jj] == ss[i]) & (jj < i))
            rank = rank + lt.astype(jnp.int32)
        oh = (k8 == rank).astype(jnp.float32)
        vals = vals + vs[i] * oh
        idxs = idxs + c_f32(cs[i]) * oh

    ps = ps_ref[...]
    slots = jax.nn.sigmoid(ps * TEMP)
    factor = jnp.sum(slots, axis=1, keepdims=True) + 1e-10
    w = slots * n_p / factor
    act_sum = jnp.sum(vals * w, axis=1, keepdims=True)
    act_out[...] = act_sum
    mind_out[...] = n_p - act_sum
    idx_out[...] = idxs


def c_f32(v):
    return v.astype(jnp.float32)


def _matcher(xt, tm3, q, ps2, pblk, interpret=False):
    b, d, t = xt.shape
    pn, _ = q.shape
    n_p = ps2.shape[1]
    p = pn // n_p
    nblk = p // pblk
    kern = functools.partial(_matcher_kernel, pblk=pblk, n_p=n_p, t=t,
                             radius=RADIUS)
    return pl.pallas_call(
        kern,
        grid=(b, nblk),
        in_specs=[
            pl.BlockSpec((None, d, t), lambda bi, ji: (bi, 0, 0)),
            pl.BlockSpec((None, 1, t), lambda bi, ji: (bi, 0, 0)),
            pl.BlockSpec((pblk * n_p, d), lambda bi, ji: (ji, 0)),
            pl.BlockSpec((1, n_p), lambda bi, ji: (0, 0)),
        ],
        out_specs=[
            pl.BlockSpec((None, None, pblk, 1), lambda bi, ji: (bi, ji, 0, 0)),
            pl.BlockSpec((None, None, pblk, 1), lambda bi, ji: (bi, ji, 0, 0)),
            pl.BlockSpec((None, None, pblk, n_p),
                         lambda bi, ji: (bi, ji, 0, 0)),
        ],
        out_shape=[
            jax.ShapeDtypeStruct((b, nblk, pblk, 1), jnp.float32),
            jax.ShapeDtypeStruct((b, nblk, pblk, 1), jnp.float32),
            jax.ShapeDtypeStruct((b, nblk, pblk, n_p), jnp.float32),
        ],
        scratch_shapes=[pltpu.VMEM((d, t), jnp.float32)],
        interpret=interpret,
    )(xt, tm3, q, ps2)


def kernel(x, timeline_mask, prototype_vectors, patch_select):
    b, t, d = x.shape
    p, _, n_p = prototype_vectors.shape
    xt = jnp.swapaxes(x, 1, 2)
    q = jnp.swapaxes(prototype_vectors, 1, 2).reshape(p * n_p, d)
    tm3 = timeline_mask[:, None, :]
    ps2 = patch_select.reshape(1, n_p)
    pblk = 16 if p % 16 == 0 else 8
    act, mind, idx = _matcher(xt, tm3, q, ps2, pblk)
    return (act.reshape(b, p), mind.reshape(b, p), idx.reshape(b, p, n_p))
